# baseline (speedup 1.0000x reference)
import functools

import jax
import jax.numpy as jnp
from jax.experimental import pallas as pl
from jax.experimental.pallas import tpu as pltpu

_Z, _Y, _X = 32, 360, 480
_M = _Z * _Y * _X


def _conv_body(g_ref, w_ref, o_ref, acc_ref):
    k = pl.program_id(1)

    @pl.when(k == 0)
    def _():
        acc_ref[...] = jnp.zeros_like(acc_ref)

    acc_ref[...] += jnp.dot(g_ref[0], w_ref[0],
                            preferred_element_type=jnp.float32)

    @pl.when(k == 26)
    def _():
        o_ref[...] = acc_ref[...]


def _stats_body(x_ref, s_ref, q_ref):
    r = pl.program_id(0)
    x = x_ref[...]

    @pl.when(r == 0)
    def _():
        s_ref[...] = jnp.zeros_like(s_ref)
        q_ref[...] = jnp.zeros_like(q_ref)

    s_ref[...] += jnp.sum(x, axis=0, keepdims=True)
    q_ref[...] += jnp.sum(x * x, axis=0, keepdims=True)


def _bn_body(x_ref, s_ref, q_ref, gamma_ref, beta_ref, o_ref, *, n):
    x = x_ref[...]
    mean = s_ref[...] / n
    var = q_ref[...] / n - mean * mean
    h = (x - mean) / jnp.sqrt(var + 1e-5) * gamma_ref[...][None, :] + beta_ref[...][None, :]
    o_ref[...] = jnp.where(h >= 0, h, 0.01 * h)


def kernel(features, voxel_indices, W, gamma, beta):
    n = features.shape[0]
    c_out = W.shape[-1]
    v = voxel_indices.astype(jnp.int32)
    grid = jnp.full((_M,), -1, jnp.int32).at[v].set(jnp.arange(n, dtype=jnp.int32))
    z = v // (_Y * _X)
    y = (v // _X) % _Y
    x = v % _X
    nidx_list = []
    for dz in (-1, 0, 1):
        for dy in (-1, 0, 1):
            for dx in (-1, 0, 1):
                nz, ny, nx = z + dz, y + dy, x + dx
                inb = (nz >= 0) & (nz < _Z) & (ny >= 0) & (ny < _Y) & (nx >= 0) & (nx < _X)
                nflat = jnp.clip(nz * (_Y * _X) + ny * _X + nx, 0, _M - 1)
                nidx = grid[nflat]
                ok = inb & (nidx >= 0)
                nidx_list.append(jnp.where(ok, nidx, n))
    nidx_all = jnp.stack(nidx_list)
    fpad = jnp.concatenate([features, jnp.zeros((8, features.shape[1]), jnp.float32)], 0)
    G = fpad[nidx_all]

    rb = 10
    nb = n // rb
    out_conv = pl.pallas_call(
        _conv_body,
        grid=(rb, 27),
        in_specs=[
            pl.BlockSpec((1, nb, features.shape[1]), lambda r, k: (k, r, 0)),
            pl.BlockSpec((1, features.shape[1], c_out), lambda r, k: (k, 0, 0)),
        ],
        out_specs=pl.BlockSpec((nb, c_out), lambda r, k: (r, 0)),
        out_shape=jax.ShapeDtypeStruct((n, c_out), jnp.float32),
        scratch_shapes=[pltpu.VMEM((nb, c_out), jnp.float32)],
    )(G, W)

    bb = 10
    nbb = n // bb
    s, q = pl.pallas_call(
        _stats_body,
        grid=(bb,),
        in_specs=[pl.BlockSpec((nbb, c_out), lambda r: (r, 0))],
        out_specs=[
            pl.BlockSpec((1, c_out), lambda r: (0, 0)),
            pl.BlockSpec((1, c_out), lambda r: (0, 0)),
        ],
        out_shape=[
            jax.ShapeDtypeStruct((1, c_out), jnp.float32),
            jax.ShapeDtypeStruct((1, c_out), jnp.float32),
        ],
    )(out_conv)
    return pl.pallas_call(
        functools.partial(_bn_body, n=n),
        grid=(bb,),
        in_specs=[
            pl.BlockSpec((nbb, c_out), lambda r: (r, 0)),
            pl.BlockSpec((1, c_out), lambda r: (0, 0)),
            pl.BlockSpec((1, c_out), lambda r: (0, 0)),
            pl.BlockSpec((c_out,), lambda r: (0,)),
            pl.BlockSpec((c_out,), lambda r: (0,)),
        ],
        out_specs=pl.BlockSpec((nbb, c_out), lambda r: (r, 0)),
        out_shape=jax.ShapeDtypeStruct((n, c_out), jnp.float32),
    )(out_conv, s, q, gamma, beta)
